# Initial kernel scaffold; baseline (speedup 1.0000x reference)
import jax
import jax.numpy as jnp
from jax.experimental import pallas as pl


def kernel(tokens, embedding_weight):
    raise NotImplementedError("write your pallas kernel here")


# speedup vs baseline: 4.1405x; 4.1405x over previous
import functools

import jax
import jax.numpy as jnp
from jax import lax
from jax.experimental import pallas as pl
from jax.experimental.pallas import tpu as pltpu
from jax.experimental.pallas import tpu_sc as plsc


def _gather_kernel(B, D, chunk):
    info = plsc.get_sparse_core_info()
    NC, NS = info.num_cores, info.num_subcores
    NW = NC * NS
    assert B % (NW * chunk) == 0
    chunks_per_w = B // (NW * chunk)
    b_per_w = B // NW

    mesh = plsc.VectorSubcoreMesh(core_axis_name="c", subcore_axis_name="s")

    @functools.partial(
        pl.kernel,
        out_type=jax.ShapeDtypeStruct((B, D), jnp.float32),
        mesh=mesh,
        scratch_types=[
            pltpu.VMEM((chunk,), jnp.int32),
            pltpu.VMEM((chunk, D), jnp.float32),
            pltpu.SemaphoreType.DMA,
        ],
        compiler_params=pltpu.CompilerParams(use_tc_tiling_on_sc=False),
    )
    def k(idx_hbm, table_hbm, out_hbm, idx_v, rows_v, sem):
        wid = lax.axis_index("s") * NC + lax.axis_index("c")
        base = wid * b_per_w

        @pl.loop(0, chunks_per_w)
        def _(g):
            off = base + g * chunk
            pltpu.sync_copy(idx_hbm.at[pl.ds(off, chunk)], idx_v)
            pltpu.async_copy(table_hbm.at[idx_v], rows_v, sem).wait()
            pltpu.sync_copy(rows_v, out_hbm.at[pl.ds(off, chunk)])

    return k


def kernel(tokens, embedding_weight):
    B, T = tokens.shape
    V, D = embedding_weight.shape
    flat_idx = tokens.reshape(B * T).astype(jnp.int32)
    k = _gather_kernel(B * T, D, chunk=1024)
    out = k(flat_idx, embedding_weight)
    return out.reshape(B, T, D)


# speedup vs baseline: 4.2641x; 1.0298x over previous
import functools

import jax
import jax.numpy as jnp
from jax import lax
from jax.experimental import pallas as pl
from jax.experimental.pallas import tpu as pltpu
from jax.experimental.pallas import tpu_sc as plsc


def _gather_kernel(B, D, chunk):
    info = plsc.get_sparse_core_info()
    NC, NS = info.num_cores, info.num_subcores
    NW = NC * NS
    NBUF = 2
    assert B % (NW * chunk) == 0
    n = B // (NW * chunk)
    assert n >= NBUF
    b_per_w = B // NW

    mesh = plsc.VectorSubcoreMesh(core_axis_name="c", subcore_axis_name="s")

    @functools.partial(
        pl.kernel,
        out_type=jax.ShapeDtypeStruct((B, D), jnp.float32),
        mesh=mesh,
        scratch_types=[
            [pltpu.VMEM((chunk,), jnp.int32) for _ in range(NBUF)],
            [pltpu.VMEM((chunk, D), jnp.float32) for _ in range(NBUF)],
            [pltpu.SemaphoreType.DMA for _ in range(NBUF)],
            [pltpu.SemaphoreType.DMA for _ in range(NBUF)],
        ],
        compiler_params=pltpu.CompilerParams(use_tc_tiling_on_sc=False),
    )
    def k(idx_hbm, table_hbm, out_hbm, idx_v, rows_v, sem_g, sem_s):
        wid = lax.axis_index("s") * NC + lax.axis_index("c")
        base = wid * b_per_w

        for b in range(NBUF):
            pltpu.sync_copy(idx_hbm.at[pl.ds(base + b * chunk, chunk)], idx_v[b])
            pltpu.async_copy(table_hbm.at[idx_v[b]], rows_v[b], sem_g[b])

        @pl.loop(0, n, step=NBUF)
        def _(g):
            for b in range(NBUF):
                c = g + b
                off = base + c * chunk
                pltpu.make_async_copy(table_hbm.at[idx_v[b]], rows_v[b], sem_g[b]).wait()
                pltpu.async_copy(rows_v[b], out_hbm.at[pl.ds(off, chunk)], sem_s[b])

                @pl.when(c + NBUF < n)
                def _():
                    off2 = base + (c + NBUF) * chunk
                    pltpu.sync_copy(idx_hbm.at[pl.ds(off2, chunk)], idx_v[b])

                pltpu.make_async_copy(rows_v[b], out_hbm.at[pl.ds(off, chunk)], sem_s[b]).wait()

                @pl.when(c + NBUF < n)
                def _():
                    pltpu.async_copy(table_hbm.at[idx_v[b]], rows_v[b], sem_g[b])

    return k


def kernel(tokens, embedding_weight):
    B, T = tokens.shape
    V, D = embedding_weight.shape
    flat_idx = tokens.reshape(B * T).astype(jnp.int32)
    k = _gather_kernel(B * T, D, chunk=800)
    out = k(flat_idx, embedding_weight)
    return out.reshape(B, T, D)


# speedup vs baseline: 4.2649x; 1.0002x over previous
import functools

import jax
import jax.numpy as jnp
from jax import lax
from jax.experimental import pallas as pl
from jax.experimental.pallas import tpu as pltpu
from jax.experimental.pallas import tpu_sc as plsc


def _gather_kernel(B, T, D, rows_per_chunk):
    info = plsc.get_sparse_core_info()
    NC, NS = info.num_cores, info.num_subcores
    NW = NC * NS
    NBUF = 2
    R = rows_per_chunk
    chunk = R * T
    assert B % (NW * R) == 0
    n = B // (NW * R)
    assert n >= NBUF
    rows_w = B // NW

    mesh = plsc.VectorSubcoreMesh(core_axis_name="c", subcore_axis_name="s")

    @functools.partial(
        pl.kernel,
        out_type=jax.ShapeDtypeStruct((B, T, D), jnp.float32),
        mesh=mesh,
        scratch_types=[
            [pltpu.VMEM((R, T), jnp.int32) for _ in range(NBUF)],
            [pltpu.VMEM((R, T, D), jnp.float32) for _ in range(NBUF)],
            [pltpu.SemaphoreType.DMA for _ in range(NBUF)],
            [pltpu.SemaphoreType.DMA for _ in range(NBUF)],
        ],
        compiler_params=pltpu.CompilerParams(use_tc_tiling_on_sc=False),
    )
    def k(tok_hbm, table_hbm, out_hbm, idx_v, rows_v, sem_g, sem_s):
        wid = lax.axis_index("s") * NC + lax.axis_index("c")
        base = wid * rows_w

        def stage_idx(c, b):
            pltpu.sync_copy(tok_hbm.at[pl.ds(base + c * R, R)], idx_v[b])

        def start_gather(b):
            for r in range(R):
                pltpu.async_copy(
                    table_hbm.at[idx_v[b].at[r]], rows_v[b].at[r], sem_g[b]
                )

        def wait_gather(b):
            for r in range(R):
                pltpu.make_async_copy(
                    table_hbm.at[idx_v[b].at[r]], rows_v[b].at[r], sem_g[b]
                ).wait()

        def store(c, b, wait):
            src = rows_v[b]
            dst = out_hbm.at[pl.ds(base + c * R, R)]
            if wait:
                pltpu.make_async_copy(src, dst, sem_s[b]).wait()
            else:
                pltpu.async_copy(src, dst, sem_s[b])

        for b in range(NBUF):
            stage_idx(b, b)
            start_gather(b)

        @pl.loop(0, n, step=NBUF)
        def _(g):
            for b in range(NBUF):
                c = g + b
                wait_gather(b)
                store(c, b, wait=False)

                @pl.when(c + NBUF < n)
                def _():
                    stage_idx(c + NBUF, b)

                store(c, b, wait=True)

                @pl.when(c + NBUF < n)
                def _():
                    start_gather(b)

    return k


def kernel(tokens, embedding_weight):
    B, T = tokens.shape
    V, D = embedding_weight.shape
    k = _gather_kernel(B, T, D, rows_per_chunk=4)
    return k(tokens.astype(jnp.int32), embedding_weight)
